# Initial kernel scaffold; baseline (speedup 1.0000x reference)
import functools

import jax
import jax.numpy as jnp
from jax import lax
from jax.experimental import pallas as pl
from jax.experimental.pallas import tpu as pltpu
from jax.experimental.pallas import tpu_sc as plsc

_INFO = plsc.get_sparse_core_info()
_NC = _INFO.num_cores
_NS = _INFO.num_subcores
_NW = _NC * _NS
_L = 16


def _edge_dot_sc(x, src, dst, n_edges, d):
    epw = n_edges // _NW
    k = 400 if epw % 400 == 0 else 16
    assert epw % k == 0 and k % _L == 0
    n_chunks = epw // k
    groups = k // _L

    mesh = plsc.VectorSubcoreMesh(core_axis_name="c", subcore_axis_name="s")

    @functools.partial(
        pl.kernel,
        mesh=mesh,
        out_type=jax.ShapeDtypeStruct((n_edges,), jnp.float32),
        scratch_types=[
            pltpu.VMEM((k,), jnp.int32),
            pltpu.VMEM((k,), jnp.int32),
            pltpu.VMEM((k, d), jnp.float32),
            pltpu.VMEM((k, d), jnp.float32),
            pltpu.VMEM((k,), jnp.float32),
            pltpu.SemaphoreType.DMA,
            pltpu.SemaphoreType.DMA,
        ],
    )
    def run(x_hbm, src_hbm, dst_hbm, out_hbm,
            src_v, dst_v, a_v, b_v, o_v, sem_a, sem_b):
        wid = lax.axis_index("s") * _NC + lax.axis_index("c")
        base = wid * epw

        def chunk_body(c, carry):
            cb = base + c * k
            pltpu.sync_copy(src_hbm.at[pl.ds(cb, k)], src_v)
            pltpu.sync_copy(dst_hbm.at[pl.ds(cb, k)], dst_v)
            a_cp = pltpu.async_copy(x_hbm.at[src_v], a_v, sem_a)
            b_cp = pltpu.async_copy(x_hbm.at[dst_v], b_v, sem_b)
            a_cp.wait()
            b_cp.wait()

            def group_body(g, carry2):
                eids = g * _L + lax.iota(jnp.int32, _L)

                def feat_body(f, acc):
                    fv = jnp.full((_L,), f, jnp.int32)
                    av = plsc.load_gather(a_v, [eids, fv])
                    bv = plsc.load_gather(b_v, [eids, fv])
                    return acc + av * bv

                acc = lax.fori_loop(0, d, feat_body,
                                    jnp.zeros((_L,), jnp.float32))
                o_v[pl.ds(g * _L, _L)] = acc
                return carry2

            lax.fori_loop(0, groups, group_body, 0)
            pltpu.sync_copy(o_v, out_hbm.at[pl.ds(cb, k)])
            return carry

        lax.fori_loop(0, n_chunks, chunk_body, 0)

    return run(x, src, dst)


def kernel(x, edge_index):
    n, d = x.shape
    n_edges = edge_index.shape[1]
    ei = edge_index.astype(jnp.int32)
    return _edge_dot_sc(x, ei[0], ei[1], n_edges, d)


# speedup vs baseline: 4.0879x; 4.0879x over previous
import functools

import jax
import jax.numpy as jnp
from jax import lax
from jax.experimental import pallas as pl
from jax.experimental.pallas import tpu as pltpu
from jax.experimental.pallas import tpu_sc as plsc

_INFO = plsc.get_sparse_core_info()
_NC = _INFO.num_cores
_NS = _INFO.num_subcores
_NW = _NC * _NS
_L = 16


def _edge_dot_sc(x, src, dst, n_edges, d):
    epw = n_edges // _NW
    k = 400 if epw % 400 == 0 else 16
    assert epw % k == 0 and k % _L == 0
    n_chunks = epw // k
    groups = k // _L

    mesh = plsc.VectorSubcoreMesh(core_axis_name="c", subcore_axis_name="s")

    @functools.partial(
        pl.kernel,
        mesh=mesh,
        out_type=jax.ShapeDtypeStruct((n_edges,), jnp.float32),
        compiler_params=pltpu.CompilerParams(needs_layout_passes=False),
        scratch_types=[
            pltpu.VMEM((k,), jnp.int32),
            pltpu.VMEM((k,), jnp.int32),
            pltpu.VMEM((k, d), jnp.float32),
            pltpu.VMEM((k, d), jnp.float32),
            pltpu.VMEM((k,), jnp.float32),
            pltpu.SemaphoreType.DMA,
            pltpu.SemaphoreType.DMA,
        ],
    )
    def run(x_hbm, src_hbm, dst_hbm, out_hbm,
            src_v, dst_v, a_v, b_v, o_v, sem_a, sem_b):
        wid = lax.axis_index("s") * _NC + lax.axis_index("c")
        base = wid * epw

        def chunk_body(c, carry):
            cb = base + c * k
            pltpu.sync_copy(src_hbm.at[pl.ds(cb, k)], src_v)
            pltpu.sync_copy(dst_hbm.at[pl.ds(cb, k)], dst_v)
            a_cp = pltpu.async_copy(x_hbm.at[src_v], a_v, sem_a)
            b_cp = pltpu.async_copy(x_hbm.at[dst_v], b_v, sem_b)
            a_cp.wait()
            b_cp.wait()

            lanes = lax.iota(jnp.int32, _L)
            last = jnp.full((_L,), _L - 1, jnp.int32)

            def group_body(g, carry2):
                res = jnp.zeros((_L,), jnp.float32)
                for j in range(_L):
                    e = g * _L + j
                    acc = jnp.zeros((_L,), jnp.float32)
                    for blk in range(d // _L):
                        av = a_v[e, pl.ds(blk * _L, _L)]
                        bv = b_v[e, pl.ds(blk * _L, _L)]
                        acc = acc + av * bv
                    tot = jnp.sum(acc)
                    res = jnp.where(lanes == j, tot, res)
                o_v[pl.ds(g * _L, _L)] = res
                return carry2

            lax.fori_loop(0, groups, group_body, 0)
            pltpu.sync_copy(o_v, out_hbm.at[pl.ds(cb, k)])
            return carry

        lax.fori_loop(0, n_chunks, chunk_body, 0)

    return run(x, src, dst)


def kernel(x, edge_index):
    n, d = x.shape
    n_edges = edge_index.shape[1]
    ei = edge_index.astype(jnp.int32)
    return _edge_dot_sc(x, ei[0], ei[1], n_edges, d)
